# baseline (speedup 1.0000x reference)
import functools

import jax
import jax.numpy as jnp
from jax import lax
from jax.experimental import pallas as pl
from jax.experimental.pallas import tpu as pltpu
from jax.experimental.pallas import tpu_sc as plsc

VOCAB = 1000000
EMB_DIM = 64
NUM_Y = 2
BATCH = 4096
HIST = 200

NUM_TILES = 32
BAGS_PER_TILE = BATCH // NUM_TILES
CHUNK = 104
HALF = HIST // 2
LANES = 16
VREGS_PER_ROW = EMB_DIM // LANES
NSLOTS = 4


def _sc_body(idx_hbm, table_hbm, w_hbm, b_hbm, out_hbm,
             idx_v, rows0, rows1, rows2, rows3, w_v, b_v, logit_v,
             sem0, sem1, sem2, sem3):
    wid = lax.axis_index("s") * 2 + lax.axis_index("c")
    rows = [rows0, rows1, rows2, rows3]
    sems = [sem0, sem1, sem2, sem3]

    pltpu.sync_copy(idx_hbm.at[wid], idx_v)
    pltpu.sync_copy(w_hbm, w_v)
    pltpu.sync_copy(b_hbm, b_v)

    w_regs = [[w_v[c, pl.ds(k * LANES, LANES)] for k in range(VREGS_PER_ROW)]
              for c in range(NUM_Y)]
    b_reg = b_v[...]
    inv_n = jnp.float32(1.0 / HIST)
    lane_iota = lax.iota(jnp.int32, LANES)
    lane_mask = lane_iota < NUM_Y
    b_sel = jnp.where(lane_iota == 0, b_reg[0], b_reg[1])
    perms = [lane_iota ^ s for s in (8, 4, 2, 1)]

    def lane_sum(v):
        for p in perms:
            v = v + v.at[p].get(mode="promise_in_bounds")
        return v

    def fire(bag, slot):
        pltpu.async_copy(table_hbm.at[idx_v.at[2 * bag]],
                         rows[slot].at[pl.ds(0, CHUNK)], sems[slot])
        pltpu.async_copy(table_hbm.at[idx_v.at[2 * bag + 1]],
                         rows[slot].at[pl.ds(CHUNK, CHUNK)], sems[slot])

    def drain(slot):
        for c in range(2):
            pltpu.make_async_copy(table_hbm.at[pl.ds(0, CHUNK)],
                                  rows[slot].at[pl.ds(c * CHUNK, CHUNK)],
                                  sems[slot]).wait()

    def reduce_bag(bag, rows_ref):
        zeros = tuple(jnp.zeros((LANES,), jnp.float32)
                      for _ in range(2 * VREGS_PER_ROW))

        @plsc.parallel_loop(0, HALF, 2, unroll=2, carry=zeros)
        def accs(j, a):
            new = []
            for u in range(2):
                for k in range(VREGS_PER_ROW):
                    new.append(
                        a[u * VREGS_PER_ROW + k]
                        + rows_ref[j + u, pl.ds(k * LANES, LANES)]
                        + rows_ref[j + u + CHUNK, pl.ds(k * LANES, LANES)])
            return tuple(new)

        pooled = [(accs[k] + accs[VREGS_PER_ROW + k]) * inv_n
                  for k in range(VREGS_PER_ROW)]
        reds = []
        for c in range(NUM_Y):
            prod = pooled[0] * w_regs[c][0]
            for k in range(1, VREGS_PER_ROW):
                prod = prod + pooled[k] * w_regs[c][k]
            reds.append(lane_sum(prod))
        vals = jnp.where(lane_iota == 0, reds[0], reds[1]) + b_sel
        plsc.store_scatter(logit_v, [2 * bag + lane_iota], vals,
                           mask=lane_mask)

    for i in range(NSLOTS - 1):
        fire(i, i)

    def group_body(g, carry):
        for u in range(NSLOTS):
            bag = NSLOTS * g + u
            drain(u)
            reduce_bag(bag, rows[u])
            nxt = bag + NSLOTS - 1

            @pl.when(nxt < BAGS_PER_TILE)
            def _():
                fire(nxt, (u + NSLOTS - 1) % NSLOTS)
        return carry

    lax.fori_loop(0, BAGS_PER_TILE // NSLOTS, group_body, 0)

    for i in range(2 * BAGS_PER_TILE // LANES):
        x = logit_v[pl.ds(i * LANES, LANES)]
        logit_v[pl.ds(i * LANES, LANES)] = 1.0 / (1.0 + jnp.exp(-x))
    pltpu.sync_copy(logit_v, out_hbm.at[pl.ds(wid * 2 * BAGS_PER_TILE,
                                              2 * BAGS_PER_TILE)])


@jax.jit
def _sc_call(idx, table, w, b_pad):
    run = functools.partial(
        pl.kernel,
        out_type=jax.ShapeDtypeStruct((BATCH * NUM_Y,), jnp.float32),
        mesh=plsc.VectorSubcoreMesh(core_axis_name="c", subcore_axis_name="s"),
        compiler_params=pltpu.CompilerParams(
            needs_layout_passes=False, use_tc_tiling_on_sc=False),
        scratch_types=(
            [pltpu.VMEM((2 * BAGS_PER_TILE, CHUNK), jnp.int32)]
            + [pltpu.VMEM((2 * CHUNK, EMB_DIM), jnp.float32)
               for _ in range(NSLOTS)]
            + [pltpu.VMEM((NUM_Y, EMB_DIM), jnp.float32),
               pltpu.VMEM((LANES,), jnp.float32),
               pltpu.VMEM((2 * BAGS_PER_TILE,), jnp.float32)]
            + [pltpu.SemaphoreType.DMA for _ in range(NSLOTS)]
        ),
    )(_sc_body)
    return run(idx, table, w, b_pad)


def kernel(input, emb_weight, W, b):
    idx = input.astype(jnp.int32).reshape(NUM_TILES, 2 * BAGS_PER_TILE, HALF)
    idx = jnp.pad(idx, ((0, 0), (0, 0), (0, CHUNK - HALF)))
    b_pad = jnp.pad(b.astype(jnp.float32), (0, LANES - NUM_Y))
    out_flat = _sc_call(idx, emb_weight, W.astype(jnp.float32), b_pad)
    return out_flat.reshape(BATCH, NUM_Y)


# speedup vs baseline: 1.9121x; 1.9121x over previous
import functools

import jax
import jax.numpy as jnp
from jax import lax
from jax.experimental import pallas as pl
from jax.experimental.pallas import tpu as pltpu
from jax.experimental.pallas import tpu_sc as plsc

VOCAB = 1000000
EMB_DIM = 64
NUM_Y = 2
BATCH = 4096
HIST = 200

NUM_TILES = 32
BAGS_PER_TILE = BATCH // NUM_TILES
CHUNK = 104
HALF = HIST // 2
LANES = 16
VREGS_PER_ROW = EMB_DIM // LANES
NSLOTS = 4


def _sc_body(idx_hbm, table_hbm, w_hbm, b_hbm, out_hbm,
             idx_v, rows0, rows1, rows2, rows3, w_v, b_v, logit_v,
             sem0, sem1, sem2, sem3):
    wid = lax.axis_index("s") * 2 + lax.axis_index("c")
    rows = [rows0, rows1, rows2, rows3]
    sems = [sem0, sem1, sem2, sem3]

    pltpu.sync_copy(idx_hbm.at[wid], idx_v)
    pltpu.sync_copy(w_hbm, w_v)
    pltpu.sync_copy(b_hbm, b_v)

    w_regs = [[w_v[c, pl.ds(k * LANES, LANES)] for k in range(VREGS_PER_ROW)]
              for c in range(NUM_Y)]
    b_reg = b_v[...]
    inv_n = jnp.float32(1.0 / HIST)
    lane_iota = lax.iota(jnp.int32, LANES)
    lane_mask = lane_iota < NUM_Y
    b_sel = jnp.where(lane_iota == 0, b_reg[0], b_reg[1])
    perms = [lane_iota ^ s for s in (8, 4, 2, 1)]

    def lane_sum(v):
        for p in perms:
            v = v + v.at[p].get(mode="promise_in_bounds")
        return v

    def fire(bag, slot):
        pltpu.async_copy(table_hbm.at[idx_v.at[2 * bag]],
                         rows[slot].at[pl.ds(0, CHUNK)], sems[slot])
        pltpu.async_copy(table_hbm.at[idx_v.at[2 * bag + 1]],
                         rows[slot].at[pl.ds(CHUNK, CHUNK)], sems[slot])

    def drain(slot):
        for c in range(2):
            pltpu.make_async_copy(table_hbm.at[pl.ds(0, CHUNK)],
                                  rows[slot].at[pl.ds(c * CHUNK, CHUNK)],
                                  sems[slot]).wait()

    def reduce_bag(bag, rows_ref):
        zeros = tuple(jnp.zeros((LANES,), jnp.float32)
                      for _ in range(2 * VREGS_PER_ROW))

        @plsc.parallel_loop(0, HALF, 2, unroll=2, carry=zeros)
        def accs(j, a):
            new = []
            for u in range(2):
                for k in range(VREGS_PER_ROW):
                    new.append(
                        a[u * VREGS_PER_ROW + k]
                        + rows_ref[j + u, pl.ds(k * LANES, LANES)]
                        + rows_ref[j + u + CHUNK, pl.ds(k * LANES, LANES)])
            return tuple(new)

        pooled = [(accs[k] + accs[VREGS_PER_ROW + k]) * inv_n
                  for k in range(VREGS_PER_ROW)]
        reds = []
        for c in range(NUM_Y):
            prod = pooled[0] * w_regs[c][0]
            for k in range(1, VREGS_PER_ROW):
                prod = prod + pooled[k] * w_regs[c][k]
            reds.append(lane_sum(prod))
        vals = jnp.where(lane_iota == 0, reds[0], reds[1]) + b_sel
        plsc.store_scatter(logit_v, [2 * bag + lane_iota], vals,
                           mask=lane_mask)

    for i in range(NSLOTS - 1):
        fire(i, i)

    def group_body(g, carry):
        for u in range(NSLOTS):
            bag = NSLOTS * g + u
            drain(u)
            reduce_bag(bag, rows[u])
            nxt = bag + NSLOTS - 1

            @pl.when(nxt < BAGS_PER_TILE)
            def _():
                fire(nxt, (u + NSLOTS - 1) % NSLOTS)
        return carry

    lax.fori_loop(0, BAGS_PER_TILE // NSLOTS, group_body, 0)

    for i in range(2 * BAGS_PER_TILE // LANES):
        x = logit_v[pl.ds(i * LANES, LANES)]
        logit_v[pl.ds(i * LANES, LANES)] = 1.0 / (1.0 + jnp.exp(-x))
    pltpu.sync_copy(logit_v, out_hbm.at[pl.ds(wid * 2 * BAGS_PER_TILE,
                                              2 * BAGS_PER_TILE)])


@jax.jit
def _sc_call(idx, table, w, b_pad):
    run = functools.partial(
        pl.kernel,
        out_type=jax.ShapeDtypeStruct((BATCH * NUM_Y,), jnp.float32),
        mesh=plsc.VectorSubcoreMesh(core_axis_name="c", subcore_axis_name="s"),
        compiler_params=pltpu.CompilerParams(
            needs_layout_passes=False, use_tc_tiling_on_sc=False),
        scratch_types=(
            [pltpu.VMEM((2 * BAGS_PER_TILE, CHUNK), jnp.int32)]
            + [pltpu.VMEM((2 * CHUNK, EMB_DIM), jnp.float32)
               for _ in range(NSLOTS)]
            + [pltpu.VMEM((NUM_Y, EMB_DIM), jnp.float32),
               pltpu.VMEM((LANES,), jnp.float32),
               pltpu.VMEM((2 * BAGS_PER_TILE,), jnp.float32)]
            + [pltpu.SemaphoreType.DMA for _ in range(NSLOTS)]
        ),
    )(_sc_body)
    return run(idx, table, w, b_pad)


def kernel(input, emb_weight, W, b):
    idx = input.astype(jnp.int32).reshape(NUM_TILES, 2 * BAGS_PER_TILE, HALF)
    idx = jnp.concatenate([idx, idx[:, :, : CHUNK - HALF]], axis=-1)
    b_pad = jnp.pad(b.astype(jnp.float32), (0, LANES - NUM_Y))
    out_flat = _sc_call(idx, emb_weight, W.astype(jnp.float32), b_pad)
    return out_flat.reshape(BATCH, NUM_Y)
